# baseline (speedup 1.0000x reference)
import jax
import jax.numpy as jnp
from jax import lax
from jax.experimental import pallas as pl
from jax.experimental.pallas import tpu as pltpu
from jax.experimental.pallas import tpu_sc as plsc

VOCAB = 1000000
EMBED = 64
NW = 32
CHUNKS = VOCAB // 128
TAILV = VOCAB - CHUNKS * 128
PITCH = 133


def _iota16():
    return lax.broadcasted_iota(jnp.int32, (16,), 0)


def _zeros16():
    return jnp.zeros((16,), jnp.int32)


def _transpose_chunk(in_ref, out_ref):
    cvecs = [_iota16() + cg * 16 for cg in range(8)]
    z16 = _zeros16()

    @plsc.parallel_loop(0, 64, unroll=4)
    def _(f):
        fvec = z16 + f
        for cg in range(8):
            v = in_ref[f, pl.ds(cg * 16, 16)]
            plsc.store_scatter(out_ref, [cvecs[cg], fvec], v)


def _k1_body(tableT, tail, tp, in0, in1, out0, out1, tail_v, tail_s, rsem, wsem):
    wid = lax.axis_index("s") * 2 + lax.axis_index("c")
    npairs = CHUNKS // (2 * NW)
    has_extra = wid < (CHUNKS - 2 * npairs * NW)

    def col0(i):
        return (wid + NW * i) * 128

    def fire_read(i, buf):
        pltpu.async_copy(tableT.at[:, pl.ds(col0(i), 128)], buf, rsem)

    def drain_read(buf):
        pltpu.make_async_copy(tableT.at[:, pl.ds(0, 128)], buf, rsem).wait()

    def fire_write(i, buf):
        pltpu.async_copy(
            buf.at[:, pl.ds(0, 128)], tp.at[pl.ds(col0(i), 128)], wsem
        )

    def drain_write(buf):
        pltpu.make_async_copy(
            buf.at[:, pl.ds(0, 128)], tp.at[pl.ds(0, 128)], wsem
        ).wait()

    fire_read(0, in0)

    def pair(q, c):
        fire_read(2 * q + 1, in1)
        drain_read(in0)

        @pl.when(q > 0)
        def _():
            drain_write(out0)

        _transpose_chunk(in0, out0)
        fire_write(2 * q, out0)

        @pl.when(2 * q + 2 < 2 * npairs)
        def _():
            fire_read(2 * q + 2, in0)

        @pl.when(jnp.logical_and(2 * q + 2 == 2 * npairs, has_extra))
        def _():
            fire_read(2 * npairs, in0)

        drain_read(in1)

        @pl.when(q > 0)
        def _():
            drain_write(out1)

        _transpose_chunk(in1, out1)
        fire_write(2 * q + 1, out1)
        return c

    lax.fori_loop(0, npairs, pair, 0, unroll=False)

    @pl.when(has_extra)
    def _():
        drain_read(in0)
        drain_write(out0)
        _transpose_chunk(in0, out0)
        fire_write(2 * npairs, out0)

    drain_write(out0)
    drain_write(out1)

    @pl.when(wid == NW - 1)
    def _():
        pltpu.sync_copy(tail, tail_v)
        for i in range(TAILV):
            for fg in range(4):
                tail_s[i, pl.ds(fg * 16, 16)] = tail_v[
                    pl.ds(i * 64 + fg * 16, 16)
                ]
        pltpu.sync_copy(tail_s, tp.at[pl.ds(CHUNKS * 128, TAILV)])


def _k2_body(idxT, tp2, out, idx_l, idx_f, rows0, rows1, ov0, ov1, gsem, wsem):
    wid = lax.axis_index("s") * 2 + lax.axis_index("c")

    def load_idx(k, kp):
        pltpu.sync_copy(
            idxT.at[pl.ds(8 * k, 8), pl.ds(wid * 128, 128)], idx_l
        )
        base = kp * 1024
        for r in range(8):
            for g in range(8):
                idx_f[pl.ds(base + r * 128 + g * 16, 16)] = idx_l[
                    r, pl.ds(g * 16, 16)
                ]

    def fire_gather(m, buf):
        kp = lax.bitwise_and(lax.shift_right_logical(m, 3), 1)
        j = lax.bitwise_and(m, 7)
        pltpu.async_copy(
            tp2.at[idx_f.at[pl.ds(kp * 1024 + j * 128, 128)]], buf, gsem
        )

    def drain_gather(buf):
        pltpu.make_async_copy(tp2.at[pl.ds(0, 128)], buf, gsem).wait()

    def transpose_block(rin, rout):
        fvecs = [_iota16() + fg * 16 for fg in range(4)]
        z16 = _zeros16()

        @plsc.parallel_loop(0, 128, unroll=8)
        def _(ln):
            lvec = z16 + ln
            for fg in range(4):
                v = rin[ln, pl.ds(fg * 16, 16)]
                plsc.store_scatter(rout, [fvecs[fg], lvec], v)

    def fire_writes(m, buf):
        rowoff = m * 2048 + wid * 8
        for tr in range(8):
            pltpu.async_copy(
                buf.at[pl.ds(tr * 8, 8), pl.ds(0, 128)],
                out.at[pl.ds(rowoff + tr * 256, 8)],
                wsem,
            )

    def drain_writes(buf):
        pltpu.make_async_copy(
            buf.at[:, pl.ds(0, 128)], out.at[pl.ds(0, 64)], wsem
        ).wait()

    load_idx(0, 0)
    fire_gather(0, rows0)

    def pair(p, c):
        fire_gather(2 * p + 1, rows1)
        drain_gather(rows0)

        @pl.when(p > 0)
        def _():
            drain_writes(ov0)

        transpose_block(rows0, ov0)

        @pl.when(lax.bitwise_and(2 * p + 2, 7) == 0)
        def _():
            knext = lax.shift_right_logical(2 * p + 2, 3)
            load_idx(knext, lax.bitwise_and(knext, 1))

        @pl.when(p < 99)
        def _():
            fire_gather(2 * p + 2, rows0)

        fire_writes(2 * p, ov0)
        drain_gather(rows1)

        @pl.when(p > 0)
        def _():
            drain_writes(ov1)

        transpose_block(rows1, ov1)
        fire_writes(2 * p + 1, ov1)
        return c

    lax.fori_loop(0, 100, pair, 0, unroll=False)
    drain_writes(ov0)
    drain_writes(ov1)


def kernel(indices, table):
    b, s = indices.shape
    mesh = plsc.VectorSubcoreMesh(core_axis_name="c", subcore_axis_name="s")

    idxT = indices.astype(jnp.int32).T
    tableT = table.T
    tail = table[CHUNKS * 128:].reshape(TAILV * 64)

    k1 = pl.kernel(
        _k1_body,
        out_type=jax.ShapeDtypeStruct((VOCAB, 128), jnp.float32),
        mesh=mesh,
        scratch_types=[
            pltpu.VMEM((64, 128), jnp.float32),
            pltpu.VMEM((64, 128), jnp.float32),
            pltpu.VMEM((128, PITCH), jnp.float32),
            pltpu.VMEM((128, PITCH), jnp.float32),
            pltpu.VMEM((TAILV * 64,), jnp.float32),
            pltpu.VMEM((TAILV, 128), jnp.float32),
            pltpu.SemaphoreType.DMA,
            pltpu.SemaphoreType.DMA,
        ],
        compiler_params=pltpu.CompilerParams(needs_layout_passes=False),
    )
    tp2 = k1(tableT, tail)

    k2 = pl.kernel(
        _k2_body,
        out_type=jax.ShapeDtypeStruct((b * s * EMBED // 128, 128), jnp.float32),
        mesh=mesh,
        scratch_types=[
            pltpu.VMEM((8, 128), jnp.int32),
            pltpu.VMEM((2048,), jnp.int32),
            pltpu.VMEM((128, 128), jnp.float32),
            pltpu.VMEM((128, 128), jnp.float32),
            pltpu.VMEM((64, PITCH), jnp.float32),
            pltpu.VMEM((64, PITCH), jnp.float32),
            pltpu.SemaphoreType.DMA,
            pltpu.SemaphoreType.DMA,
        ],
        compiler_params=pltpu.CompilerParams(needs_layout_passes=False),
    )
    out2 = k2(idxT, tp2)
    out5 = out2.reshape(s, 8, b // 128, 8, 128)
    return out5.transpose(2, 4, 0, 1, 3).reshape(b, s, EMBED)


# speedup vs baseline: 1.0897x; 1.0897x over previous
import jax
import jax.numpy as jnp
from jax import lax
from jax.experimental import pallas as pl
from jax.experimental.pallas import tpu as pltpu
from jax.experimental.pallas import tpu_sc as plsc

VOCAB = 1000000
EMBED = 64
NW = 32
CHUNKS = VOCAB // 128
TAILV = VOCAB - CHUNKS * 128
PITCH = 133


def _iota16():
    return lax.broadcasted_iota(jnp.int32, (16,), 0)


def _zeros16():
    return jnp.zeros((16,), jnp.int32)


def _transpose_chunk(in_ref, out_ref):
    fvecs = [_iota16() + fg * 16 for fg in range(4)]
    z16 = _zeros16()

    @plsc.parallel_loop(0, 128, unroll=8)
    def _(c):
        cvec = z16 + c
        for fg in range(4):
            v = plsc.load_gather(in_ref, [fvecs[fg], cvec])
            out_ref[c, pl.ds(fg * 16, 16)] = v


def _k1_body(tableT, tail, tp, in0, in1, out0, out1, tail_v, tail_s, rsem, wsem):
    wid = lax.axis_index("s") * 2 + lax.axis_index("c")
    npairs = CHUNKS // (2 * NW)
    has_extra = wid < (CHUNKS - 2 * npairs * NW)

    def col0(i):
        return (wid + NW * i) * 128

    def fire_read(i, buf):
        pltpu.async_copy(tableT.at[:, pl.ds(col0(i), 128)], buf, rsem)

    def drain_read(buf):
        pltpu.make_async_copy(tableT.at[:, pl.ds(0, 128)], buf, rsem).wait()

    def fire_write(i, buf):
        pltpu.async_copy(buf, tp.at[pl.ds(col0(i), 128)], wsem)

    def drain_write(buf):
        pltpu.make_async_copy(buf, tp.at[pl.ds(0, 128)], wsem).wait()

    fire_read(0, in0)

    def pair(q, c):
        fire_read(2 * q + 1, in1)
        drain_read(in0)

        @pl.when(q > 0)
        def _():
            drain_write(out0)

        _transpose_chunk(in0, out0)
        fire_write(2 * q, out0)

        @pl.when(2 * q + 2 < 2 * npairs)
        def _():
            fire_read(2 * q + 2, in0)

        @pl.when(jnp.logical_and(2 * q + 2 == 2 * npairs, has_extra))
        def _():
            fire_read(2 * npairs, in0)

        drain_read(in1)

        @pl.when(q > 0)
        def _():
            drain_write(out1)

        _transpose_chunk(in1, out1)
        fire_write(2 * q + 1, out1)
        return c

    lax.fori_loop(0, npairs, pair, 0, unroll=False)

    @pl.when(has_extra)
    def _():
        drain_read(in0)
        drain_write(out0)
        _transpose_chunk(in0, out0)
        fire_write(2 * npairs, out0)

    drain_write(out0)
    drain_write(out1)

    @pl.when(wid == NW - 1)
    def _():
        pltpu.sync_copy(tail, tail_v)
        for i in range(TAILV):
            for fg in range(4):
                tail_s[i, pl.ds(fg * 16, 16)] = tail_v[
                    pl.ds(i * 64 + fg * 16, 16)
                ]
        pltpu.sync_copy(tail_s, tp.at[pl.ds(CHUNKS * 128, TAILV)])


def _k2_body(idxT, tp2, out, idx_l, idx_f, rows0, rows1, ov0, ov1, gsem, wsem):
    wid = lax.axis_index("s") * 2 + lax.axis_index("c")

    def load_idx(k, kp):
        pltpu.sync_copy(
            idxT.at[pl.ds(8 * k, 8), pl.ds(wid * 128, 128)], idx_l
        )
        base = kp * 1024
        for r in range(8):
            for g in range(8):
                idx_f[pl.ds(base + r * 128 + g * 16, 16)] = idx_l[
                    r, pl.ds(g * 16, 16)
                ]

    def fire_gather(m, buf):
        kp = lax.bitwise_and(lax.shift_right_logical(m, 3), 1)
        j = lax.bitwise_and(m, 7)
        pltpu.async_copy(
            tp2.at[idx_f.at[pl.ds(kp * 1024 + j * 128, 128)]], buf, gsem
        )

    def drain_gather(buf):
        pltpu.make_async_copy(tp2.at[pl.ds(0, 128)], buf, gsem).wait()

    def transpose_block(rin, rout):
        lvecs = [_iota16() + lg * 16 for lg in range(8)]
        z16 = _zeros16()

        @plsc.parallel_loop(0, 64, unroll=4)
        def _(f):
            fvec = z16 + f
            for lg in range(8):
                v = plsc.load_gather(rin, [lvecs[lg], fvec])
                rout[f, pl.ds(lg * 16, 16)] = v

    def fire_writes(m, buf):
        rowoff = m * 2048 + wid * 8
        for tr in range(8):
            pltpu.async_copy(
                buf.at[pl.ds(tr * 8, 8)],
                out.at[pl.ds(rowoff + tr * 256, 8)],
                wsem,
            )

    def drain_writes(buf):
        pltpu.make_async_copy(buf, out.at[pl.ds(0, 64)], wsem).wait()

    load_idx(0, 0)
    fire_gather(0, rows0)

    def pair(p, c):
        fire_gather(2 * p + 1, rows1)
        drain_gather(rows0)

        @pl.when(p > 0)
        def _():
            drain_writes(ov0)

        transpose_block(rows0, ov0)

        @pl.when(lax.bitwise_and(2 * p + 2, 7) == 0)
        def _():
            knext = lax.shift_right_logical(2 * p + 2, 3)
            load_idx(knext, lax.bitwise_and(knext, 1))

        @pl.when(p < 99)
        def _():
            fire_gather(2 * p + 2, rows0)

        fire_writes(2 * p, ov0)
        drain_gather(rows1)

        @pl.when(p > 0)
        def _():
            drain_writes(ov1)

        transpose_block(rows1, ov1)
        fire_writes(2 * p + 1, ov1)
        return c

    lax.fori_loop(0, 100, pair, 0, unroll=False)
    drain_writes(ov0)
    drain_writes(ov1)


def kernel(indices, table):
    b, s = indices.shape
    mesh = plsc.VectorSubcoreMesh(core_axis_name="c", subcore_axis_name="s")

    idxT = indices.astype(jnp.int32).T
    tableT = table.T
    tail = table[CHUNKS * 128:].reshape(TAILV * 64)

    k1 = pl.kernel(
        _k1_body,
        out_type=jax.ShapeDtypeStruct((VOCAB, 128), jnp.float32),
        mesh=mesh,
        scratch_types=[
            pltpu.VMEM((64, 128), jnp.float32),
            pltpu.VMEM((64, 128), jnp.float32),
            pltpu.VMEM((128, 128), jnp.float32),
            pltpu.VMEM((128, 128), jnp.float32),
            pltpu.VMEM((TAILV * 64,), jnp.float32),
            pltpu.VMEM((TAILV, 128), jnp.float32),
            pltpu.SemaphoreType.DMA,
            pltpu.SemaphoreType.DMA,
        ],
        compiler_params=pltpu.CompilerParams(needs_layout_passes=False),
    )
    tp2 = k1(tableT, tail)

    k2 = pl.kernel(
        _k2_body,
        out_type=jax.ShapeDtypeStruct((b * s * EMBED // 128, 128), jnp.float32),
        mesh=mesh,
        scratch_types=[
            pltpu.VMEM((8, 128), jnp.int32),
            pltpu.VMEM((2048,), jnp.int32),
            pltpu.VMEM((128, 128), jnp.float32),
            pltpu.VMEM((128, 128), jnp.float32),
            pltpu.VMEM((64, 128), jnp.float32),
            pltpu.VMEM((64, 128), jnp.float32),
            pltpu.SemaphoreType.DMA,
            pltpu.SemaphoreType.DMA,
        ],
        compiler_params=pltpu.CompilerParams(needs_layout_passes=False),
    )
    out2 = k2(idxT, tp2)
    out5 = out2.reshape(s, 8, b // 128, 8, 128)
    return out5.transpose(2, 4, 0, 1, 3).reshape(b, s, EMBED)


# speedup vs baseline: 1.2791x; 1.1738x over previous
import functools

import jax
import jax.numpy as jnp
from jax import lax
from jax.experimental import pallas as pl
from jax.experimental.pallas import tpu as pltpu
from jax.experimental.pallas import tpu_sc as plsc

EMBED = 64
NC = 2
NS = 16
NW = NC * NS
CH = 128
JJ = 4
GR = CH * JJ


def _emb_body(tot, idx_hbm, table_hbm, out_hbm, idx_v, rows_v, gsem):
    pw = tot // NW
    ng = pw // GR
    wid = lax.axis_index("s") * NC + lax.axis_index("c")
    row0 = wid * pw

    def fire(g, b):
        pltpu.sync_copy(idx_hbm.at[pl.ds(row0 + g * GR, GR)], idx_v.at[b])
        for j in range(JJ):
            pltpu.async_copy(
                table_hbm.at[idx_v.at[b, pl.ds(j * CH, CH)]],
                rows_v.at[b, pl.ds(j * CH, CH)],
                gsem,
            )

    def drain(b):
        pltpu.make_async_copy(
            table_hbm.at[pl.ds(0, GR)], rows_v.at[b], gsem
        ).wait()

    def store(g, b):
        pltpu.sync_copy(rows_v.at[b], out_hbm.at[pl.ds(row0 + g * GR, GR)])

    fire(0, 0)

    def body(p, carry):
        g0 = p * 2
        fire(g0 + 1, 1)
        drain(0)
        store(g0, 0)
        fire(g0 + 2, 0)
        drain(1)
        store(g0 + 1, 1)
        return carry

    lax.fori_loop(0, ng // 2 - 1, body, 0, unroll=False)

    g0 = ng - 2
    fire(g0 + 1, 1)
    drain(0)
    store(g0, 0)
    drain(1)
    store(g0 + 1, 1)


def kernel(indices, table):
    b, s = indices.shape
    tot = b * s
    idx_flat = indices.astype(jnp.int32).reshape(tot)
    grid_kernel = pl.kernel(
        functools.partial(_emb_body, tot),
        out_type=jax.ShapeDtypeStruct((tot, EMBED), jnp.float32),
        mesh=plsc.VectorSubcoreMesh(core_axis_name="c", subcore_axis_name="s"),
        scratch_types=[
            pltpu.VMEM((2, GR), jnp.int32),
            pltpu.VMEM((2, GR, EMBED), jnp.float32),
            pltpu.SemaphoreType.DMA,
        ],
        compiler_params=pltpu.CompilerParams(use_tc_tiling_on_sc=False),
    )
    out = grid_kernel(idx_flat, table)
    return out.reshape(b, s, EMBED)


# speedup vs baseline: 1.5716x; 1.2286x over previous
import functools

import jax
import jax.numpy as jnp
from jax import lax
from jax.experimental import pallas as pl
from jax.experimental.pallas import tpu as pltpu
from jax.experimental.pallas import tpu_sc as plsc

VOCAB = 1000000
EMBED = 64
NW = 32
GR = 256
CH = 128


def _body(tot, idx_hbm, table_hbm, out_hbm, idx0, idx1, rows0, rows1, cb0, gsem):
    pw = tot // NW
    ng = pw // GR
    wid = lax.axis_index("s") * 2 + lax.axis_index("c")
    row0 = wid * pw

    def fire(g, ibuf, rbuf):
        pltpu.sync_copy(idx_hbm.at[pl.ds(row0 + g * GR, GR)], ibuf)
        for j in range(GR // CH):
            pltpu.async_copy(
                table_hbm.at[ibuf.at[pl.ds(j * CH, CH)]],
                rbuf.at[pl.ds(j * CH, CH)],
                gsem,
            )

    def drain(rbuf):
        pltpu.make_async_copy(table_hbm.at[pl.ds(0, GR)], rbuf, gsem).wait()

    def store(g, rbuf, cbuf):
        @plsc.parallel_loop(0, GR, unroll=8)
        def _(r):
            for fg in range(4):
                cbuf[r, pl.ds(fg * 16, 16)] = rbuf[r, pl.ds(fg * 16, 16)]

        pltpu.sync_copy(cbuf, out_hbm.at[pl.ds(row0 + g * GR, GR)])

    fire(0, idx0, rows0)

    def body(p, carry):
        g0 = p * 2
        fire(g0 + 1, idx1, rows1)
        drain(rows0)
        store(g0, rows0, cb0)
        fire(g0 + 2, idx0, rows0)
        drain(rows1)
        store(g0 + 1, rows1, cb0)
        return carry

    lax.fori_loop(0, ng // 2 - 1, body, 0, unroll=False)

    g0 = ng - 2
    fire(g0 + 1, idx1, rows1)
    drain(rows0)
    store(g0, rows0, cb0)
    drain(rows1)
    store(g0 + 1, rows1, cb0)


def kernel(indices, table):
    b, s = indices.shape
    tot = b * s
    idx_flat = indices.astype(jnp.int32).reshape(tot)
    table_p = jnp.pad(table, ((0, 0), (0, 128 - EMBED)))

    grid_kernel = pl.kernel(
        functools.partial(_body, tot),
        out_type=jax.ShapeDtypeStruct((tot, EMBED), jnp.float32),
        mesh=plsc.VectorSubcoreMesh(core_axis_name="c", subcore_axis_name="s"),
        scratch_types=[
            pltpu.VMEM((GR,), jnp.int32),
            pltpu.VMEM((GR,), jnp.int32),
            pltpu.VMEM((GR, 128), jnp.float32),
            pltpu.VMEM((GR, 128), jnp.float32),
            pltpu.VMEM((GR, EMBED), jnp.float32),
            pltpu.SemaphoreType.DMA,
        ],
        compiler_params=pltpu.CompilerParams(needs_layout_passes=False),
    )
    out = grid_kernel(idx_flat, table_p)
    return out.reshape(b, s, EMBED)
